# Initial kernel scaffold; baseline (speedup 1.0000x reference)
import jax
import jax.numpy as jnp
from jax.experimental import pallas as pl


def kernel(x, edge_index):
    raise NotImplementedError("write your pallas kernel here")


# speedup vs baseline: 4.9167x; 4.9167x over previous
import dataclasses
import functools

import jax
import jax.numpy as jnp
from jax import lax
from jax.experimental import pallas as pl
from jax.experimental.pallas import tpu as pltpu
from jax.experimental.pallas import tpu_sc as plsc

NC = 2
NS = 16
NW = NC * NS
LANES = 16


def _make_kernel(n_nodes, feat, n_edges, chunk):
    per_tile = n_edges // NW
    n_chunks = per_tile // chunk
    mesh = plsc.VectorSubcoreMesh(core_axis_name="c", subcore_axis_name="s")
    cp = pltpu.CompilerParams()
    if "needs_layout_passes" in pltpu.CompilerParams.__dataclass_fields__:
        cp = dataclasses.replace(cp, needs_layout_passes=False)

    @functools.partial(
        pl.kernel,
        mesh=mesh,
        compiler_params=cp,
        out_type=jax.ShapeDtypeStruct((n_edges,), jnp.float32),
        scratch_types=[
            pltpu.VMEM((chunk,), jnp.int32),
            pltpu.VMEM((chunk,), jnp.int32),
            pltpu.VMEM((chunk, feat), jnp.float32),
            pltpu.VMEM((chunk, feat), jnp.float32),
            pltpu.VMEM((chunk,), jnp.float32),
            pltpu.VMEM((LANES * LANES,), jnp.float32),
            pltpu.SemaphoreType.DMA,
            pltpu.SemaphoreType.DMA,
        ],
    )
    def k(x_hbm, src_hbm, dst_hbm, out_hbm,
          idx_s, idx_d, rows_s, rows_d, out_v, part_v, sem_s, sem_d):
        wid = lax.axis_index("s") * NC + lax.axis_index("c")
        tile_base = wid * per_tile
        col_idx = lax.iota(jnp.int32, LANES) * LANES

        @pl.loop(0, n_chunks)
        def _(j):
            base = tile_base + j * chunk
            pltpu.sync_copy(src_hbm.at[pl.ds(base, chunk)], idx_s)
            pltpu.sync_copy(dst_hbm.at[pl.ds(base, chunk)], idx_d)
            cp_s = pltpu.async_copy(x_hbm.at[idx_s], rows_s, sem_s)
            cp_d = pltpu.async_copy(x_hbm.at[idx_d], rows_d, sem_d)
            cp_s.wait()
            cp_d.wait()

            @pl.loop(0, chunk // LANES)
            def _(g):
                e0 = g * LANES
                for el in range(LANES):
                    e = e0 + el
                    acc = rows_s[e, pl.ds(0, LANES)] * rows_d[e, pl.ds(0, LANES)]
                    for c in range(1, feat // LANES):
                        acc = acc + (rows_s[e, pl.ds(c * LANES, LANES)]
                                     * rows_d[e, pl.ds(c * LANES, LANES)])
                    plsc.store_scatter(part_v, [col_idx + el], acc)
                tot = part_v[pl.ds(0, LANES)]
                for i in range(1, LANES):
                    tot = tot + part_v[pl.ds(i * LANES, LANES)]
                out_v[pl.ds(e0, LANES)] = tot

            pltpu.sync_copy(out_v, out_hbm.at[pl.ds(base, chunk)])

    return k


def kernel(x, edge_index):
    n_nodes, feat = x.shape
    n_edges = edge_index.shape[1]
    src = edge_index[0].astype(jnp.int32)
    dst = edge_index[1].astype(jnp.int32)
    k = _make_kernel(n_nodes, feat, n_edges, chunk=400)
    return k(x, src, dst)


# speedup vs baseline: 8.0106x; 1.6293x over previous
import dataclasses
import functools

import jax
import jax.numpy as jnp
from jax import lax
from jax.experimental import pallas as pl
from jax.experimental.pallas import tpu as pltpu
from jax.experimental.pallas import tpu_sc as plsc

NC = 2
NS = 16
NW = NC * NS
LANES = 16


def _make_kernel(n_nodes, feat, n_edges, chunk):
    per_tile = n_edges // NW
    n_chunks = per_tile // chunk
    assert per_tile % chunk == 0 and chunk % LANES == 0 and chunk % 8 == 0
    assert n_chunks % 2 == 1
    mesh = plsc.VectorSubcoreMesh(core_axis_name="c", subcore_axis_name="s")
    cp = pltpu.CompilerParams()
    if "needs_layout_passes" in pltpu.CompilerParams.__dataclass_fields__:
        cp = dataclasses.replace(cp, needs_layout_passes=False)

    @functools.partial(
        pl.kernel,
        mesh=mesh,
        compiler_params=cp,
        out_type=jax.ShapeDtypeStruct((n_edges,), jnp.float32),
        scratch_types=[
            pltpu.VMEM((per_tile,), jnp.int32),
            pltpu.VMEM((per_tile,), jnp.int32),
            pltpu.VMEM((chunk, feat), jnp.float32),
            pltpu.VMEM((chunk, feat), jnp.float32),
            pltpu.VMEM((chunk, feat), jnp.float32),
            pltpu.VMEM((chunk, feat), jnp.float32),
            pltpu.VMEM((per_tile,), jnp.float32),
            pltpu.VMEM((LANES * LANES,), jnp.float32),
            pltpu.SemaphoreType.DMA,
            pltpu.SemaphoreType.DMA,
            pltpu.SemaphoreType.DMA,
            pltpu.SemaphoreType.DMA,
        ],
    )
    def k(x_hbm, src_hbm, dst_hbm, out_hbm,
          idx_s, idx_d, rs0, rd0, rs1, rd1, out_v, part_v,
          sem_s0, sem_d0, sem_s1, sem_d1):
        wid = lax.axis_index("s") * NC + lax.axis_index("c")
        tile_base = wid * per_tile
        col_idx = lax.iota(jnp.int32, LANES) * LANES

        pltpu.sync_copy(src_hbm.at[pl.ds(tile_base, per_tile)], idx_s)
        pltpu.sync_copy(dst_hbm.at[pl.ds(tile_base, per_tile)], idx_d)

        def issue(j, rs, rd, sem_s, sem_d):
            pltpu.async_copy(
                x_hbm.at[idx_s.at[pl.ds(j * chunk, chunk)]], rs, sem_s)
            pltpu.async_copy(
                x_hbm.at[idx_d.at[pl.ds(j * chunk, chunk)]], rd, sem_d)

        def wait(rs, rd, sem_s, sem_d):
            pltpu.make_async_copy(x_hbm.at[pl.ds(0, chunk)], rs, sem_s).wait()
            pltpu.make_async_copy(x_hbm.at[pl.ds(0, chunk)], rd, sem_d).wait()

        def compute(j, rows_s, rows_d):
            base = j * chunk

            @pl.loop(0, chunk // LANES)
            def _(g):
                e0 = g * LANES
                for el in range(LANES):
                    e = e0 + el
                    acc = rows_s[e, pl.ds(0, LANES)] * rows_d[e, pl.ds(0, LANES)]
                    for c in range(1, feat // LANES):
                        acc = acc + (rows_s[e, pl.ds(c * LANES, LANES)]
                                     * rows_d[e, pl.ds(c * LANES, LANES)])
                    plsc.store_scatter(part_v, [col_idx + el], acc)
                tot = part_v[pl.ds(0, LANES)]
                for i in range(1, LANES):
                    tot = tot + part_v[pl.ds(i * LANES, LANES)]
                out_v[pl.ds(base + e0, LANES)] = tot

        issue(0, rs0, rd0, sem_s0, sem_d0)

        @pl.loop(0, (n_chunks - 1) // 2)
        def _(jj):
            j = jj * 2
            issue(j + 1, rs1, rd1, sem_s1, sem_d1)
            wait(rs0, rd0, sem_s0, sem_d0)
            compute(j, rs0, rd0)
            issue(j + 2, rs0, rd0, sem_s0, sem_d0)
            wait(rs1, rd1, sem_s1, sem_d1)
            compute(j + 1, rs1, rd1)

        wait(rs0, rd0, sem_s0, sem_d0)
        compute(n_chunks - 1, rs0, rd0)

        pltpu.sync_copy(out_v, out_hbm.at[pl.ds(tile_base, per_tile)])

    return k


def kernel(x, edge_index):
    n_nodes, feat = x.shape
    n_edges = edge_index.shape[1]
    src = edge_index[0].astype(jnp.int32)
    dst = edge_index[1].astype(jnp.int32)
    k = _make_kernel(n_nodes, feat, n_edges, chunk=80)
    return k(x, src, dst)


# speedup vs baseline: 8.1266x; 1.0145x over previous
import dataclasses
import functools

import jax
import jax.numpy as jnp
from jax import lax
from jax.experimental import pallas as pl
from jax.experimental.pallas import tpu as pltpu
from jax.experimental.pallas import tpu_sc as plsc

NC = 2
NS = 16
NW = NC * NS
LANES = 16


def _make_kernel(n_nodes, feat, n_edges, chunk):
    per_tile = n_edges // NW
    n_chunks = per_tile // chunk
    assert per_tile % chunk == 0 and chunk % LANES == 0 and chunk % 8 == 0
    assert n_chunks % 2 == 1
    mesh = plsc.VectorSubcoreMesh(core_axis_name="c", subcore_axis_name="s")
    cp = pltpu.CompilerParams()
    for field, val in (("needs_layout_passes", False),
                       ("use_tc_tiling_on_sc", False)):
        if field in pltpu.CompilerParams.__dataclass_fields__:
            cp = dataclasses.replace(cp, **{field: val})

    @functools.partial(
        pl.kernel,
        mesh=mesh,
        compiler_params=cp,
        out_type=jax.ShapeDtypeStruct((n_edges,), jnp.float32),
        scratch_types=[
            pltpu.VMEM((per_tile,), jnp.int32),
            pltpu.VMEM((per_tile,), jnp.int32),
            pltpu.VMEM((chunk, feat // 2), jnp.int32),
            pltpu.VMEM((chunk, feat // 2), jnp.int32),
            pltpu.VMEM((chunk, feat // 2), jnp.int32),
            pltpu.VMEM((chunk, feat // 2), jnp.int32),
            pltpu.VMEM((per_tile,), jnp.float32),
            pltpu.VMEM((LANES * LANES,), jnp.float32),
            pltpu.SemaphoreType.DMA,
            pltpu.SemaphoreType.DMA,
            pltpu.SemaphoreType.DMA,
            pltpu.SemaphoreType.DMA,
        ],
    )
    def k(x_hbm, src_hbm, dst_hbm, out_hbm,
          idx_s, idx_d, rs0, rd0, rs1, rd1, out_v, part_v,
          sem_s0, sem_d0, sem_s1, sem_d1):
        wid = lax.axis_index("s") * NC + lax.axis_index("c")
        tile_base = wid * per_tile
        col_idx = lax.iota(jnp.int32, LANES) * LANES

        pltpu.sync_copy(src_hbm.at[pl.ds(tile_base, per_tile)], idx_s)
        pltpu.sync_copy(dst_hbm.at[pl.ds(tile_base, per_tile)], idx_d)

        def issue(j, rs, rd, sem_s, sem_d):
            pltpu.async_copy(
                x_hbm.at[idx_s.at[pl.ds(j * chunk, chunk)]], rs, sem_s)
            pltpu.async_copy(
                x_hbm.at[idx_d.at[pl.ds(j * chunk, chunk)]], rd, sem_d)

        def wait(rs, rd, sem_s, sem_d):
            pltpu.make_async_copy(x_hbm.at[pl.ds(0, chunk)], rs, sem_s).wait()
            pltpu.make_async_copy(x_hbm.at[pl.ds(0, chunk)], rd, sem_d).wait()

        def compute(j, rows_s, rows_d):
            base = j * chunk

            @pl.loop(0, chunk // LANES)
            def _(g):
                e0 = g * LANES
                for el in range(LANES):
                    e = e0 + el
                    m = [plsc.bitcast(rows_s[e, pl.ds(c * LANES, LANES)],
                                      jnp.bfloat16)
                         * plsc.bitcast(rows_d[e, pl.ds(c * LANES, LANES)],
                                        jnp.bfloat16)
                         for c in range(feat // (2 * LANES))]
                    while len(m) > 1:
                        m = [m[i] + m[i + 1] for i in range(0, len(m), 2)]
                    lo, hi = plsc.unpack(m[0], format=plsc.PackFormat.INTERLEAVED)
                    acc = lo + hi
                    plsc.store_scatter(part_v, [col_idx + el], acc)
                tot = part_v[pl.ds(0, LANES)]
                for i in range(1, LANES):
                    tot = tot + part_v[pl.ds(i * LANES, LANES)]
                out_v[pl.ds(base + e0, LANES)] = tot

        issue(0, rs0, rd0, sem_s0, sem_d0)

        @pl.loop(0, (n_chunks - 1) // 2)
        def _(jj):
            j = jj * 2
            issue(j + 1, rs1, rd1, sem_s1, sem_d1)
            wait(rs0, rd0, sem_s0, sem_d0)
            compute(j, rs0, rd0)
            issue(j + 2, rs0, rd0, sem_s0, sem_d0)
            wait(rs1, rd1, sem_s1, sem_d1)
            compute(j + 1, rs1, rd1)

        wait(rs0, rd0, sem_s0, sem_d0)
        compute(n_chunks - 1, rs0, rd0)

        pltpu.sync_copy(out_v, out_hbm.at[pl.ds(tile_base, per_tile)])

    return k


def kernel(x, edge_index):
    n_nodes, feat = x.shape
    n_edges = edge_index.shape[1]
    src = edge_index[0].astype(jnp.int32)
    dst = edge_index[1].astype(jnp.int32)
    k = _make_kernel(n_nodes, feat, n_edges, chunk=80)
    xb = x.astype(jnp.bfloat16).reshape(n_nodes, feat // 2, 2)
    xi = jax.lax.bitcast_convert_type(xb, jnp.int32)
    return k(xi, src, dst)


# speedup vs baseline: 10.1963x; 1.2547x over previous
import dataclasses
import functools

import jax
import jax.numpy as jnp
from jax import lax
from jax.experimental import pallas as pl
from jax.experimental.pallas import tpu as pltpu
from jax.experimental.pallas import tpu_sc as plsc

NC = 2
NS = 16
NW = NC * NS
LANES = 16


def _make_kernel(n_nodes, feat, n_edges, chunk):
    per_tile = n_edges // NW
    n_chunks = per_tile // chunk
    assert per_tile % chunk == 0 and chunk % LANES == 0 and chunk % 8 == 0
    assert n_chunks % 2 == 1
    mesh = plsc.VectorSubcoreMesh(core_axis_name="c", subcore_axis_name="s")
    cp = pltpu.CompilerParams()
    for field, val in (("needs_layout_passes", False),
                       ("use_tc_tiling_on_sc", False)):
        if field in pltpu.CompilerParams.__dataclass_fields__:
            cp = dataclasses.replace(cp, **{field: val})

    @functools.partial(
        pl.kernel,
        mesh=mesh,
        compiler_params=cp,
        out_type=jax.ShapeDtypeStruct((n_edges,), jnp.float32),
        scratch_types=[
            pltpu.VMEM((per_tile,), jnp.int32),
            pltpu.VMEM((per_tile,), jnp.int32),
            pltpu.VMEM((chunk, feat // 2), jnp.int32),
            pltpu.VMEM((chunk, feat // 2), jnp.int32),
            pltpu.VMEM((chunk, feat // 2), jnp.int32),
            pltpu.VMEM((chunk, feat // 2), jnp.int32),
            pltpu.VMEM((per_tile,), jnp.float32),
            pltpu.VMEM((LANES * LANES,), jnp.float32),
            pltpu.VMEM((LANES * LANES,), jnp.float32),
            pltpu.SemaphoreType.DMA,
            pltpu.SemaphoreType.DMA,
            pltpu.SemaphoreType.DMA,
            pltpu.SemaphoreType.DMA,
        ],
    )
    def k(x_hbm, src_hbm, dst_hbm, out_hbm,
          idx_s, idx_d, rs0, rd0, rs1, rd1, out_v, part_a, part_b,
          sem_s0, sem_d0, sem_s1, sem_d1):
        wid = lax.axis_index("s") * NC + lax.axis_index("c")
        tile_base = wid * per_tile
        col_idx = lax.iota(jnp.int32, LANES) * LANES

        pltpu.sync_copy(src_hbm.at[pl.ds(tile_base, per_tile)], idx_s)
        pltpu.sync_copy(dst_hbm.at[pl.ds(tile_base, per_tile)], idx_d)

        def issue(j, rs, rd, sem_s, sem_d):
            pltpu.async_copy(
                x_hbm.at[idx_s.at[pl.ds(j * chunk, chunk)]], rs, sem_s)
            pltpu.async_copy(
                x_hbm.at[idx_d.at[pl.ds(j * chunk, chunk)]], rd, sem_d)

        def wait(rs, rd, sem_s, sem_d):
            pltpu.make_async_copy(x_hbm.at[pl.ds(0, chunk)], rs, sem_s).wait()
            pltpu.make_async_copy(x_hbm.at[pl.ds(0, chunk)], rd, sem_d).wait()

        def do_group(base, g, rows_s, rows_d, part):
            e0 = g * LANES
            accs = []
            for el in range(LANES):
                e = e0 + el
                m = [plsc.bitcast(rows_s[e, pl.ds(c * LANES, LANES)],
                                  jnp.bfloat16)
                     * plsc.bitcast(rows_d[e, pl.ds(c * LANES, LANES)],
                                    jnp.bfloat16)
                     for c in range(feat // (2 * LANES))]
                while len(m) > 1:
                    m = [m[i] + m[i + 1] for i in range(0, len(m), 2)]
                lo, hi = plsc.unpack(m[0], format=plsc.PackFormat.INTERLEAVED)
                accs.append(lo + hi)
            for el in range(LANES):
                plsc.store_scatter(part, [col_idx + el], accs[el])
            rows = [part[pl.ds(i * LANES, LANES)] for i in range(LANES)]
            while len(rows) > 1:
                rows = [rows[i] + rows[i + 1] for i in range(0, len(rows), 2)]
            out_v[pl.ds(base + e0, LANES)] = rows[0]

        def compute(j, rows_s, rows_d):
            base = j * chunk
            n_groups = chunk // LANES

            @pl.loop(0, n_groups // 2)
            def _(i):
                do_group(base, i * 2, rows_s, rows_d, part_a)
                do_group(base, i * 2 + 1, rows_s, rows_d, part_b)

            if n_groups % 2:
                do_group(base, n_groups - 1, rows_s, rows_d, part_a)

        issue(0, rs0, rd0, sem_s0, sem_d0)

        @pl.loop(0, (n_chunks - 1) // 2)
        def _(jj):
            j = jj * 2
            issue(j + 1, rs1, rd1, sem_s1, sem_d1)
            wait(rs0, rd0, sem_s0, sem_d0)
            compute(j, rs0, rd0)
            issue(j + 2, rs0, rd0, sem_s0, sem_d0)
            wait(rs1, rd1, sem_s1, sem_d1)
            compute(j + 1, rs1, rd1)

        wait(rs0, rd0, sem_s0, sem_d0)
        compute(n_chunks - 1, rs0, rd0)

        pltpu.sync_copy(out_v, out_hbm.at[pl.ds(tile_base, per_tile)])

    return k


def kernel(x, edge_index):
    n_nodes, feat = x.shape
    n_edges = edge_index.shape[1]
    src = edge_index[0].astype(jnp.int32)
    dst = edge_index[1].astype(jnp.int32)
    k = _make_kernel(n_nodes, feat, n_edges, chunk=80)
    xb = x.astype(jnp.bfloat16).reshape(n_nodes, feat // 2, 2)
    xi = jax.lax.bitcast_convert_type(xb, jnp.int32)
    return k(xi, src, dst)


# speedup vs baseline: 10.2103x; 1.0014x over previous
import dataclasses
import functools

import jax
import jax.numpy as jnp
from jax import lax
from jax.experimental import pallas as pl
from jax.experimental.pallas import tpu as pltpu
from jax.experimental.pallas import tpu_sc as plsc

NC = 2
NS = 16
NW = NC * NS
LANES = 16


def _make_kernel(n_nodes, feat, n_edges, chunk):
    per_tile = n_edges // NW
    n_chunks = per_tile // chunk
    assert per_tile % chunk == 0 and chunk % LANES == 0 and chunk % 8 == 0
    assert n_chunks % 2 == 1
    mesh = plsc.VectorSubcoreMesh(core_axis_name="c", subcore_axis_name="s")
    cp = pltpu.CompilerParams()
    for field, val in (("needs_layout_passes", False),
                       ("use_tc_tiling_on_sc", False),
                       ("disable_bounds_checks", True),
                       ("disable_semaphore_checks", True)):
        if field in pltpu.CompilerParams.__dataclass_fields__:
            cp = dataclasses.replace(cp, **{field: val})

    @functools.partial(
        pl.kernel,
        mesh=mesh,
        compiler_params=cp,
        out_type=jax.ShapeDtypeStruct((n_edges,), jnp.float32),
        scratch_types=[
            pltpu.VMEM((per_tile,), jnp.int32),
            pltpu.VMEM((per_tile,), jnp.int32),
            pltpu.VMEM((chunk, feat // 2), jnp.int32),
            pltpu.VMEM((chunk, feat // 2), jnp.int32),
            pltpu.VMEM((chunk, feat // 2), jnp.int32),
            pltpu.VMEM((chunk, feat // 2), jnp.int32),
            pltpu.VMEM((per_tile,), jnp.float32),
            pltpu.VMEM((LANES * LANES,), jnp.float32),
            pltpu.VMEM((LANES * LANES,), jnp.float32),
            pltpu.SemaphoreType.DMA,
            pltpu.SemaphoreType.DMA,
            pltpu.SemaphoreType.DMA,
            pltpu.SemaphoreType.DMA,
        ],
    )
    def k(x_hbm, src_hbm, dst_hbm, out_hbm,
          idx_s, idx_d, rs0, rd0, rs1, rd1, out_v, part_a, part_b,
          sem_s0, sem_d0, sem_s1, sem_d1):
        wid = lax.axis_index("s") * NC + lax.axis_index("c")
        tile_base = wid * per_tile
        col_idx = lax.iota(jnp.int32, LANES) * LANES

        pltpu.sync_copy(src_hbm.at[pl.ds(tile_base, per_tile)], idx_s)
        pltpu.sync_copy(dst_hbm.at[pl.ds(tile_base, per_tile)], idx_d)

        def issue(j, rs, rd, sem_s, sem_d):
            pltpu.async_copy(
                x_hbm.at[idx_s.at[pl.ds(j * chunk, chunk)]], rs, sem_s)
            pltpu.async_copy(
                x_hbm.at[idx_d.at[pl.ds(j * chunk, chunk)]], rd, sem_d)

        def wait(rs, rd, sem_s, sem_d):
            pltpu.make_async_copy(x_hbm.at[pl.ds(0, chunk)], rs, sem_s).wait()
            pltpu.make_async_copy(x_hbm.at[pl.ds(0, chunk)], rd, sem_d).wait()

        def do_group(base, g, rows_s, rows_d, part):
            e0 = g * LANES
            accs = []
            for el in range(LANES):
                e = e0 + el
                m = [plsc.bitcast(rows_s[e, pl.ds(c * LANES, LANES)],
                                  jnp.bfloat16)
                     * plsc.bitcast(rows_d[e, pl.ds(c * LANES, LANES)],
                                    jnp.bfloat16)
                     for c in range(feat // (2 * LANES))]
                while len(m) > 1:
                    m = [m[i] + m[i + 1] for i in range(0, len(m), 2)]
                lo, hi = plsc.unpack(m[0], format=plsc.PackFormat.INTERLEAVED)
                accs.append(lo + hi)
            for el in range(LANES):
                plsc.store_scatter(part, [col_idx + el], accs[el])
            rows = [part[pl.ds(i * LANES, LANES)] for i in range(LANES)]
            while len(rows) > 1:
                rows = [rows[i] + rows[i + 1] for i in range(0, len(rows), 2)]
            out_v[pl.ds(base + e0, LANES)] = rows[0]

        def compute(j, rows_s, rows_d):
            base = j * chunk
            n_groups = chunk // LANES

            @pl.loop(0, n_groups // 2)
            def _(i):
                do_group(base, i * 2, rows_s, rows_d, part_a)
                do_group(base, i * 2 + 1, rows_s, rows_d, part_b)

            if n_groups % 2:
                do_group(base, n_groups - 1, rows_s, rows_d, part_a)

        issue(0, rs0, rd0, sem_s0, sem_d0)

        @pl.loop(0, (n_chunks - 1) // 2)
        def _(jj):
            j = jj * 2
            issue(j + 1, rs1, rd1, sem_s1, sem_d1)
            wait(rs0, rd0, sem_s0, sem_d0)
            compute(j, rs0, rd0)
            issue(j + 2, rs0, rd0, sem_s0, sem_d0)
            wait(rs1, rd1, sem_s1, sem_d1)
            compute(j + 1, rs1, rd1)

        wait(rs0, rd0, sem_s0, sem_d0)
        compute(n_chunks - 1, rs0, rd0)

        pltpu.sync_copy(out_v, out_hbm.at[pl.ds(tile_base, per_tile)])

    return k


def kernel(x, edge_index):
    n_nodes, feat = x.shape
    n_edges = edge_index.shape[1]
    src = edge_index[0].astype(jnp.int32)
    dst = edge_index[1].astype(jnp.int32)
    k = _make_kernel(n_nodes, feat, n_edges, chunk=80)
    xb = x.astype(jnp.bfloat16).reshape(n_nodes, feat // 2, 2)
    xi = jax.lax.bitcast_convert_type(xb, jnp.int32)
    return k(xi, src, dst)
